# baseline (speedup 1.0000x reference)
import functools

import jax
import jax.numpy as jnp
from jax import lax
from jax.experimental import pallas as pl
from jax.experimental.pallas import tpu as pltpu
from jax.experimental.pallas import tpu_sc as plsc

_N = 10000
_E = 320000
_D = 128
_G = 64
_NC = 2
_NS = 16
_NW = _NC * _NS
_CH = 25
_EPT = 10000
_RPT = 624
_LAST = _N - _RPT * (_NS - 1)
_PADROWS = 16


_NCHUNK = _EPT // _CH
_NBUF = 8
_SEG = 16
_NSEG = _NCHUNK // _SEG


def _spmm_body(x_hbm, edge_hbm, zero_hbm, out_hbm,
               sidx, didx, rows, agg, semg, sems, semi):
    cid = lax.axis_index("c")
    sid = lax.axis_index("s")
    wid = cid * _NS + sid
    base = sid * _RPT
    src_hbm = edge_hbm.at[0]
    dst_hbm = edge_hbm.at[1]

    @pl.when(sid < _NS - 1)
    def _():
        pltpu.sync_copy(zero_hbm.at[pl.ds(base, _RPT)],
                        agg.at[pl.ds(base, _RPT)])

    @pl.when(sid == _NS - 1)
    def _():
        pltpu.sync_copy(zero_hbm.at[pl.ds(_RPT * (_NS - 1), _LAST + _PADROWS)],
                        agg.at[pl.ds(_RPT * (_NS - 1), _LAST + _PADROWS)])

    plsc.subcore_barrier()

    def gather(k, b, s):
        pltpu.async_copy(x_hbm.at[sidx[s].at[k]], rows[b], semg[b])

    def gather_wait(b):
        pltpu.make_async_copy(x_hbm.at[sidx[0].at[0]], rows[b], semg[b]).wait()

    def scatter(k, b, s):
        pltpu.async_copy(rows[b], agg.at[didx[s].at[k]], sems[b], add=True)

    def scatter_wait(b):
        pltpu.make_async_copy(rows[b], agg.at[didx[0].at[0]], sems[b]).wait()

    def load_idx(seg, s):
        segbase = wid * _NCHUNK + seg * _SEG
        pltpu.async_copy(src_hbm.at[pl.ds(segbase, _SEG)], sidx[s], semi[s])
        pltpu.async_copy(dst_hbm.at[pl.ds(segbase, _SEG)], didx[s], semi[s])

    def wait_idx(s):
        pltpu.make_async_copy(src_hbm.at[pl.ds(0, _SEG)], sidx[s], semi[s]).wait()
        pltpu.make_async_copy(dst_hbm.at[pl.ds(0, _SEG)], didx[s], semi[s]).wait()

    _NITER = _SEG // _NBUF

    load_idx(0, 0)
    wait_idx(0)
    for b in range(_NBUF):
        gather(b, b, 0)

    for seg in range(_NSEG):
        s = seg % 2
        ns = (seg + 1) % 2
        if seg + 1 < _NSEG:
            load_idx(seg + 1, ns)

        def step(i, carry):
            k0 = i * _NBUF
            for b in range(_NBUF):
                gather_wait(b)
                scatter(k0 + b, b, s)
            for b in range(_NBUF):
                @pl.when(i < _NITER - 1)
                def _(b=b):
                    scatter_wait(b)
                    gather(k0 + _NBUF + b, b, s)
            return carry

        lax.fori_loop(0, _NITER, step, 0)

        if seg + 1 < _NSEG:
            wait_idx(ns)
            for b in range(_NBUF):
                scatter_wait(b)
                gather(b, b, ns)
        else:
            for b in range(_NBUF):
                scatter_wait(b)

    plsc.subcore_barrier()

    @pl.when(jnp.logical_and(cid == 0, sid < _NS - 1))
    def _():
        pltpu.sync_copy(agg.at[pl.ds(base, _RPT)],
                        out_hbm.at[0].at[pl.ds(base, _RPT)])

    @pl.when(jnp.logical_and(cid == 0, sid == _NS - 1))
    def _():
        pltpu.sync_copy(agg.at[pl.ds(_RPT * (_NS - 1), _LAST)],
                        out_hbm.at[0].at[pl.ds(_RPT * (_NS - 1), _LAST)])

    @pl.when(jnp.logical_and(cid == 1, sid < _NS - 1))
    def _():
        pltpu.sync_copy(agg.at[pl.ds(base, _RPT)],
                        out_hbm.at[1].at[pl.ds(base, _RPT)])

    @pl.when(jnp.logical_and(cid == 1, sid == _NS - 1))
    def _():
        pltpu.sync_copy(agg.at[pl.ds(_RPT * (_NS - 1), _LAST)],
                        out_hbm.at[1].at[pl.ds(_RPT * (_NS - 1), _LAST)])


@functools.cache
def _get_spmm_call():
    return pl.kernel(
        _spmm_body,
        out_type=jax.ShapeDtypeStruct((_NC, _N, _D), jnp.float32),
        mesh=plsc.VectorSubcoreMesh(core_axis_name="c", subcore_axis_name="s",
                                    num_cores=_NC, num_subcores=_NS),
        scratch_types=[
            [pltpu.VMEM((_SEG, _CH), jnp.int32) for _ in range(2)],
            [pltpu.VMEM((_SEG, _CH), jnp.int32) for _ in range(2)],
            [pltpu.VMEM((_CH, _D), jnp.float32) for _ in range(_NBUF)],
            pltpu.VMEM_SHARED((_N + _PADROWS, _D), jnp.float32),
            [pltpu.SemaphoreType.DMA for _ in range(_NBUF)],
            [pltpu.SemaphoreType.DMA for _ in range(_NBUF)],
            [pltpu.SemaphoreType.DMA for _ in range(2)],
        ],
    )


def _spmm_call(x, edge3, zero_rows):
    return _get_spmm_call()(x, edge3, zero_rows)


def _mlp_body(x_ref, a0_ref, a1_ref, w1_ref, b1_ref, w2_ref, b2_ref, o_ref):
    h = (x_ref[...] + a0_ref[...].reshape(_BR, _D)
         + a1_ref[...].reshape(_BR, _D))
    h = jnp.maximum(
        jnp.dot(h, w1_ref[...], preferred_element_type=jnp.float32)
        + b1_ref[...], 0.0)
    h = jnp.maximum(
        jnp.dot(h, w2_ref[...], preferred_element_type=jnp.float32)
        + b2_ref[...], 0.0)
    o_ref[...] = h


_BR = 2000
_NB = _N // _BR


def _mlp_tc(x, agg, w1, b1, w2, b2):
    row_spec = pl.BlockSpec((_BR, _D), lambda i: (i, 0))
    a0_spec = pl.BlockSpec((1, _BR, _D), lambda i: (0, i, 0))
    a1_spec = pl.BlockSpec((1, _BR, _D), lambda i: (1, i, 0))
    w_spec = pl.BlockSpec((_D, _D), lambda i: (0, 0))
    b_spec = pl.BlockSpec((1, _D), lambda i: (0, 0))
    return pl.pallas_call(
        _mlp_body,
        grid=(_NB,),
        in_specs=[row_spec, a0_spec, a1_spec, w_spec, b_spec, w_spec, b_spec],
        out_specs=row_spec,
        out_shape=jax.ShapeDtypeStruct((_N, _D), jnp.float32),
    )(x, agg, agg, w1, b1, w2, b2)


def _final_body(x_ref, a0_ref, a1_ref, w1_ref, b1_ref, w2_ref, b2_ref,
                bt_ref, wfc_ref, bfc_ref, o_ref, p1_ref, p2_ref):
    i = pl.program_id(0)

    @pl.when(i == 0)
    def _():
        p1_ref[...] = jnp.zeros_like(p1_ref)
        p2_ref[...] = jnp.zeros_like(p2_ref)

    x1b = x_ref[...]
    h = (x1b + a0_ref[...].reshape(_BR, _D)
         + a1_ref[...].reshape(_BR, _D))
    h = jnp.maximum(
        jnp.dot(h, w1_ref[...], preferred_element_type=jnp.float32)
        + b1_ref[...], 0.0)
    x2b = jnp.maximum(
        jnp.dot(h, w2_ref[...], preferred_element_type=jnp.float32)
        + b2_ref[...], 0.0)

    b2d = bt_ref[...].reshape(1, _BR)
    gids = lax.broadcasted_iota(jnp.int32, (_G, _BR), 0)
    pt = (gids == b2d).astype(jnp.float32)
    p1_ref[...] += jnp.dot(pt, x1b, preferred_element_type=jnp.float32)
    p2_ref[...] += jnp.dot(pt, x2b, preferred_element_type=jnp.float32)

    @pl.when(i == _NB - 1)
    def _():
        pooled = (
            jnp.dot(p1_ref[...], wfc_ref[0:_D, :],
                    preferred_element_type=jnp.float32)
            + jnp.dot(p2_ref[...], wfc_ref[_D:2 * _D, :],
                      preferred_element_type=jnp.float32)
            + bfc_ref[...])
        m = jnp.max(pooled, axis=-1, keepdims=True)
        lse = jnp.log(jnp.sum(jnp.exp(pooled - m), axis=-1, keepdims=True)) + m
        o_ref[...] = pooled - lse


def _final_tc(x1, agg, w1, b1, w2, b2, batch3, wfc, bfc):
    row_spec = pl.BlockSpec((_BR, _D), lambda i: (i, 0))
    a0_spec = pl.BlockSpec((1, _BR, _D), lambda i: (0, i, 0))
    a1_spec = pl.BlockSpec((1, _BR, _D), lambda i: (1, i, 0))
    w_spec = pl.BlockSpec((_D, _D), lambda i: (0, 0))
    b_spec = pl.BlockSpec((1, _D), lambda i: (0, 0))
    return pl.pallas_call(
        _final_body,
        grid=(_NB,),
        in_specs=[
            row_spec, a0_spec, a1_spec, w_spec, b_spec, w_spec, b_spec,
            pl.BlockSpec((1, 1, _BR), lambda i: (i, 0, 0)),
            pl.BlockSpec((2 * _D, _D), lambda i: (0, 0)),
            b_spec,
        ],
        out_specs=pl.BlockSpec((_G, _D), lambda i: (0, 0)),
        out_shape=jax.ShapeDtypeStruct((_G, _D), jnp.float32),
        scratch_shapes=[
            pltpu.VMEM((_G, _D), jnp.float32),
            pltpu.VMEM((_G, _D), jnp.float32),
        ],
    )(x1, agg, agg, w1, b1, w2, b2, batch3, wfc, bfc)


def kernel(x, edge_index, batch, W1_0, b1_0, W2_0, b2_0,
           W1_1, b1_1, W2_1, b2_1, Wfc, bfc):
    edge3 = edge_index.reshape(2, _NW * _NCHUNK, _CH)
    zero_rows = jnp.zeros((_N + _PADROWS, _D), jnp.float32)

    b1_0r = b1_0.reshape(1, _D)
    b2_0r = b2_0.reshape(1, _D)
    b1_1r = b1_1.reshape(1, _D)
    b2_1r = b2_1.reshape(1, _D)
    bfcr = bfc.reshape(1, _D)
    batch3 = batch.reshape(_NB, 1, _BR)

    agg1 = _spmm_call(x, edge3, zero_rows)
    x1 = _mlp_tc(x, agg1, W1_0, b1_0r, W2_0, b2_0r)
    agg2 = _spmm_call(x1, edge3, zero_rows)
    out = _final_tc(x1, agg2, W1_1, b1_1r, W2_1, b2_1r, batch3, Wfc, bfcr)
    return out


# speedup vs baseline: 1.0851x; 1.0851x over previous
import functools

import jax
import jax.numpy as jnp
from jax import lax
from jax.experimental import pallas as pl
from jax.experimental.pallas import tpu as pltpu
from jax.experimental.pallas import tpu_sc as plsc

_N = 10000
_E = 320000
_D = 128
_G = 64
_NC = 2
_NS = 16
_NW = _NC * _NS
_CH = 50
_EPT = 10000
_RPT = 624
_LAST = _N - _RPT * (_NS - 1)
_PADROWS = 16


_NCHUNK = _EPT // _CH
_NBUF = 4
_SEG = 40
_NSEG = _NCHUNK // _SEG


def _spmm_body(x_hbm, edge_hbm, zero_hbm, out_hbm,
               sidx, didx, rows, agg, semg, sems, semi):
    cid = lax.axis_index("c")
    sid = lax.axis_index("s")
    wid = cid * _NS + sid
    base = sid * _RPT
    src_hbm = edge_hbm.at[0]
    dst_hbm = edge_hbm.at[1]

    @pl.when(sid < _NS - 1)
    def _():
        pltpu.sync_copy(zero_hbm.at[pl.ds(base, _RPT)],
                        agg.at[pl.ds(base, _RPT)])

    @pl.when(sid == _NS - 1)
    def _():
        pltpu.sync_copy(zero_hbm.at[pl.ds(_RPT * (_NS - 1), _LAST + _PADROWS)],
                        agg.at[pl.ds(_RPT * (_NS - 1), _LAST + _PADROWS)])

    plsc.subcore_barrier()

    def gather(k, b, s):
        pltpu.async_copy(x_hbm.at[sidx[s].at[k]], rows[b], semg[b])

    def gather_wait(b):
        pltpu.make_async_copy(x_hbm.at[sidx[0].at[0]], rows[b], semg[b]).wait()

    def scatter(k, b, s):
        pltpu.async_copy(rows[b], agg.at[didx[s].at[k]], sems[b], add=True)

    def scatter_wait(b):
        pltpu.make_async_copy(rows[b], agg.at[didx[0].at[0]], sems[b]).wait()

    def load_idx(seg, s):
        segbase = wid * _NCHUNK + seg * _SEG
        pltpu.async_copy(src_hbm.at[pl.ds(segbase, _SEG)], sidx[s], semi[s])
        pltpu.async_copy(dst_hbm.at[pl.ds(segbase, _SEG)], didx[s], semi[s])

    def wait_idx(s):
        pltpu.make_async_copy(src_hbm.at[pl.ds(0, _SEG)], sidx[s], semi[s]).wait()
        pltpu.make_async_copy(dst_hbm.at[pl.ds(0, _SEG)], didx[s], semi[s]).wait()

    _NITER = _SEG // _NBUF

    load_idx(0, 0)
    wait_idx(0)
    for b in range(_NBUF):
        gather(b, b, 0)

    for seg in range(_NSEG):
        s = seg % 2
        ns = (seg + 1) % 2
        if seg + 1 < _NSEG:
            load_idx(seg + 1, ns)

        def step(i, carry):
            k0 = i * _NBUF
            for b in range(_NBUF):
                gather_wait(b)
                scatter(k0 + b, b, s)
            for b in range(_NBUF):
                @pl.when(i < _NITER - 1)
                def _(b=b):
                    scatter_wait(b)
                    gather(k0 + _NBUF + b, b, s)
            return carry

        lax.fori_loop(0, _NITER, step, 0)

        if seg + 1 < _NSEG:
            wait_idx(ns)
            for b in range(_NBUF):
                scatter_wait(b)
                gather(b, b, ns)
        else:
            for b in range(_NBUF):
                scatter_wait(b)

    plsc.subcore_barrier()

    @pl.when(jnp.logical_and(cid == 0, sid < _NS - 1))
    def _():
        pltpu.sync_copy(agg.at[pl.ds(base, _RPT)],
                        out_hbm.at[0].at[pl.ds(base, _RPT)])

    @pl.when(jnp.logical_and(cid == 0, sid == _NS - 1))
    def _():
        pltpu.sync_copy(agg.at[pl.ds(_RPT * (_NS - 1), _LAST)],
                        out_hbm.at[0].at[pl.ds(_RPT * (_NS - 1), _LAST)])

    @pl.when(jnp.logical_and(cid == 1, sid < _NS - 1))
    def _():
        pltpu.sync_copy(agg.at[pl.ds(base, _RPT)],
                        out_hbm.at[1].at[pl.ds(base, _RPT)])

    @pl.when(jnp.logical_and(cid == 1, sid == _NS - 1))
    def _():
        pltpu.sync_copy(agg.at[pl.ds(_RPT * (_NS - 1), _LAST)],
                        out_hbm.at[1].at[pl.ds(_RPT * (_NS - 1), _LAST)])


@functools.cache
def _get_spmm_call():
    return pl.kernel(
        _spmm_body,
        out_type=jax.ShapeDtypeStruct((_NC, _N, _D), jnp.float32),
        mesh=plsc.VectorSubcoreMesh(core_axis_name="c", subcore_axis_name="s",
                                    num_cores=_NC, num_subcores=_NS),
        scratch_types=[
            [pltpu.VMEM((_SEG, _CH), jnp.int32) for _ in range(2)],
            [pltpu.VMEM((_SEG, _CH), jnp.int32) for _ in range(2)],
            [pltpu.VMEM((_CH, _D), jnp.float32) for _ in range(_NBUF)],
            pltpu.VMEM_SHARED((_N + _PADROWS, _D), jnp.float32),
            [pltpu.SemaphoreType.DMA for _ in range(_NBUF)],
            [pltpu.SemaphoreType.DMA for _ in range(_NBUF)],
            [pltpu.SemaphoreType.DMA for _ in range(2)],
        ],
    )


def _spmm_call(x, edge3, zero_rows):
    return _get_spmm_call()(x, edge3, zero_rows)


def _mlp_body(x_ref, a0_ref, a1_ref, w1_ref, b1_ref, w2_ref, b2_ref, o_ref):
    h = (x_ref[...] + a0_ref[...].reshape(_BR, _D)
         + a1_ref[...].reshape(_BR, _D))
    h = jnp.maximum(
        jnp.dot(h, w1_ref[...], preferred_element_type=jnp.float32)
        + b1_ref[...], 0.0)
    h = jnp.maximum(
        jnp.dot(h, w2_ref[...], preferred_element_type=jnp.float32)
        + b2_ref[...], 0.0)
    o_ref[...] = h


_BR = 2000
_NB = _N // _BR


def _mlp_tc(x, agg, w1, b1, w2, b2):
    row_spec = pl.BlockSpec((_BR, _D), lambda i: (i, 0))
    a0_spec = pl.BlockSpec((1, _BR, _D), lambda i: (0, i, 0))
    a1_spec = pl.BlockSpec((1, _BR, _D), lambda i: (1, i, 0))
    w_spec = pl.BlockSpec((_D, _D), lambda i: (0, 0))
    b_spec = pl.BlockSpec((1, _D), lambda i: (0, 0))
    return pl.pallas_call(
        _mlp_body,
        grid=(_NB,),
        in_specs=[row_spec, a0_spec, a1_spec, w_spec, b_spec, w_spec, b_spec],
        out_specs=row_spec,
        out_shape=jax.ShapeDtypeStruct((_N, _D), jnp.float32),
    )(x, agg, agg, w1, b1, w2, b2)


def _final_body(x_ref, a0_ref, a1_ref, w1_ref, b1_ref, w2_ref, b2_ref,
                bt_ref, wfc_ref, bfc_ref, o_ref, p1_ref, p2_ref):
    i = pl.program_id(0)

    @pl.when(i == 0)
    def _():
        p1_ref[...] = jnp.zeros_like(p1_ref)
        p2_ref[...] = jnp.zeros_like(p2_ref)

    x1b = x_ref[...]
    h = (x1b + a0_ref[...].reshape(_BR, _D)
         + a1_ref[...].reshape(_BR, _D))
    h = jnp.maximum(
        jnp.dot(h, w1_ref[...], preferred_element_type=jnp.float32)
        + b1_ref[...], 0.0)
    x2b = jnp.maximum(
        jnp.dot(h, w2_ref[...], preferred_element_type=jnp.float32)
        + b2_ref[...], 0.0)

    b2d = bt_ref[...].reshape(1, _BR)
    gids = lax.broadcasted_iota(jnp.int32, (_G, _BR), 0)
    pt = (gids == b2d).astype(jnp.float32)
    p1_ref[...] += jnp.dot(pt, x1b, preferred_element_type=jnp.float32)
    p2_ref[...] += jnp.dot(pt, x2b, preferred_element_type=jnp.float32)

    @pl.when(i == _NB - 1)
    def _():
        pooled = (
            jnp.dot(p1_ref[...], wfc_ref[0:_D, :],
                    preferred_element_type=jnp.float32)
            + jnp.dot(p2_ref[...], wfc_ref[_D:2 * _D, :],
                      preferred_element_type=jnp.float32)
            + bfc_ref[...])
        m = jnp.max(pooled, axis=-1, keepdims=True)
        lse = jnp.log(jnp.sum(jnp.exp(pooled - m), axis=-1, keepdims=True)) + m
        o_ref[...] = pooled - lse


def _final_tc(x1, agg, w1, b1, w2, b2, batch3, wfc, bfc):
    row_spec = pl.BlockSpec((_BR, _D), lambda i: (i, 0))
    a0_spec = pl.BlockSpec((1, _BR, _D), lambda i: (0, i, 0))
    a1_spec = pl.BlockSpec((1, _BR, _D), lambda i: (1, i, 0))
    w_spec = pl.BlockSpec((_D, _D), lambda i: (0, 0))
    b_spec = pl.BlockSpec((1, _D), lambda i: (0, 0))
    return pl.pallas_call(
        _final_body,
        grid=(_NB,),
        in_specs=[
            row_spec, a0_spec, a1_spec, w_spec, b_spec, w_spec, b_spec,
            pl.BlockSpec((1, 1, _BR), lambda i: (i, 0, 0)),
            pl.BlockSpec((2 * _D, _D), lambda i: (0, 0)),
            b_spec,
        ],
        out_specs=pl.BlockSpec((_G, _D), lambda i: (0, 0)),
        out_shape=jax.ShapeDtypeStruct((_G, _D), jnp.float32),
        scratch_shapes=[
            pltpu.VMEM((_G, _D), jnp.float32),
            pltpu.VMEM((_G, _D), jnp.float32),
        ],
    )(x1, agg, agg, w1, b1, w2, b2, batch3, wfc, bfc)


def kernel(x, edge_index, batch, W1_0, b1_0, W2_0, b2_0,
           W1_1, b1_1, W2_1, b2_1, Wfc, bfc):
    edge3 = edge_index.reshape(2, _NW * _NCHUNK, _CH)
    zero_rows = jnp.zeros((_N + _PADROWS, _D), jnp.float32)

    b1_0r = b1_0.reshape(1, _D)
    b2_0r = b2_0.reshape(1, _D)
    b1_1r = b1_1.reshape(1, _D)
    b2_1r = b2_1.reshape(1, _D)
    bfcr = bfc.reshape(1, _D)
    batch3 = batch.reshape(_NB, 1, _BR)

    agg1 = _spmm_call(x, edge3, zero_rows)
    x1 = _mlp_tc(x, agg1, W1_0, b1_0r, W2_0, b2_0r)
    agg2 = _spmm_call(x1, edge3, zero_rows)
    out = _final_tc(x1, agg2, W1_1, b1_1r, W2_1, b2_1r, batch3, Wfc, bfcr)
    return out


# speedup vs baseline: 1.1382x; 1.0489x over previous
import functools

import jax
import jax.numpy as jnp
from jax import lax
from jax.experimental import pallas as pl
from jax.experimental.pallas import tpu as pltpu
from jax.experimental.pallas import tpu_sc as plsc

_N = 10000
_E = 320000
_D = 128
_G = 64
_NC = 2
_NS = 16
_NW = _NC * _NS
_CH = 40
_EPT = 10000
_RPT = 624
_LAST = _N - _RPT * (_NS - 1)
_PADROWS = 16


_NCHUNK = _EPT // _CH
_NBUF = 5
_SEG = 50
_NSEG = _NCHUNK // _SEG


def _spmm_body(x_hbm, edge_hbm, zero_hbm, out_hbm,
               sidx, didx, rows, agg, semg, sems, semi):
    cid = lax.axis_index("c")
    sid = lax.axis_index("s")
    wid = cid * _NS + sid
    base = sid * _RPT
    src_hbm = edge_hbm.at[0]
    dst_hbm = edge_hbm.at[1]

    @pl.when(sid < _NS - 1)
    def _():
        pltpu.sync_copy(zero_hbm.at[pl.ds(base, _RPT)],
                        agg.at[pl.ds(base, _RPT)])

    @pl.when(sid == _NS - 1)
    def _():
        pltpu.sync_copy(zero_hbm.at[pl.ds(_RPT * (_NS - 1), _LAST + _PADROWS)],
                        agg.at[pl.ds(_RPT * (_NS - 1), _LAST + _PADROWS)])

    plsc.subcore_barrier()

    def gather(k, b, s):
        pltpu.async_copy(x_hbm.at[sidx[s].at[pl.ds(k * _CH, _CH)]],
                         rows[b], semg[b])

    def gather_wait(b):
        pltpu.make_async_copy(x_hbm.at[sidx[0].at[pl.ds(0, _CH)]],
                              rows[b], semg[b]).wait()

    def scatter(k, b, s):
        pltpu.async_copy(rows[b], agg.at[didx[s].at[pl.ds(k * _CH, _CH)]],
                         sems[b], add=True)

    def scatter_wait(b):
        pltpu.make_async_copy(rows[b], agg.at[didx[0].at[pl.ds(0, _CH)]],
                              sems[b]).wait()

    def load_idx(seg, s):
        segbase = wid * _EPT + seg * _SEG * _CH
        pltpu.async_copy(src_hbm.at[pl.ds(segbase, _SEG * _CH)],
                         sidx[s], semi[s])
        pltpu.async_copy(dst_hbm.at[pl.ds(segbase, _SEG * _CH)],
                         didx[s], semi[s])

    def wait_idx(s):
        pltpu.make_async_copy(src_hbm.at[pl.ds(0, _SEG * _CH)],
                              sidx[s], semi[s]).wait()
        pltpu.make_async_copy(dst_hbm.at[pl.ds(0, _SEG * _CH)],
                              didx[s], semi[s]).wait()

    _NITER = _SEG // _NBUF

    load_idx(0, 0)
    wait_idx(0)
    for b in range(_NBUF):
        gather(b, b, 0)

    for seg in range(_NSEG):
        s = seg % 2
        ns = (seg + 1) % 2
        if seg + 1 < _NSEG:
            load_idx(seg + 1, ns)

        def step(i, carry):
            k0 = i * _NBUF
            for b in range(_NBUF):
                gather_wait(b)
                scatter(k0 + b, b, s)
            for b in range(_NBUF):
                @pl.when(i < _NITER - 1)
                def _(b=b):
                    scatter_wait(b)
                    gather(k0 + _NBUF + b, b, s)
            return carry

        lax.fori_loop(0, _NITER, step, 0)

        if seg + 1 < _NSEG:
            wait_idx(ns)
            for b in range(_NBUF):
                scatter_wait(b)
                gather(b, b, ns)
        else:
            for b in range(_NBUF):
                scatter_wait(b)

    plsc.subcore_barrier()

    @pl.when(jnp.logical_and(cid == 0, sid < _NS - 1))
    def _():
        pltpu.sync_copy(agg.at[pl.ds(base, _RPT)],
                        out_hbm.at[0].at[pl.ds(base, _RPT)])

    @pl.when(jnp.logical_and(cid == 0, sid == _NS - 1))
    def _():
        pltpu.sync_copy(agg.at[pl.ds(_RPT * (_NS - 1), _LAST)],
                        out_hbm.at[0].at[pl.ds(_RPT * (_NS - 1), _LAST)])

    @pl.when(jnp.logical_and(cid == 1, sid < _NS - 1))
    def _():
        pltpu.sync_copy(agg.at[pl.ds(base, _RPT)],
                        out_hbm.at[1].at[pl.ds(base, _RPT)])

    @pl.when(jnp.logical_and(cid == 1, sid == _NS - 1))
    def _():
        pltpu.sync_copy(agg.at[pl.ds(_RPT * (_NS - 1), _LAST)],
                        out_hbm.at[1].at[pl.ds(_RPT * (_NS - 1), _LAST)])


@functools.cache
def _get_spmm_call():
    return pl.kernel(
        _spmm_body,
        out_type=jax.ShapeDtypeStruct((_NC, _N, _D), jnp.float32),
        mesh=plsc.VectorSubcoreMesh(core_axis_name="c", subcore_axis_name="s",
                                    num_cores=_NC, num_subcores=_NS),
        compiler_params=pltpu.CompilerParams(use_tc_tiling_on_sc=False),
        scratch_types=[
            [pltpu.VMEM((_SEG * _CH,), jnp.int32) for _ in range(2)],
            [pltpu.VMEM((_SEG * _CH,), jnp.int32) for _ in range(2)],
            [pltpu.VMEM((_CH, _D), jnp.float32) for _ in range(_NBUF)],
            pltpu.VMEM_SHARED((_N + _PADROWS, _D), jnp.float32),
            [pltpu.SemaphoreType.DMA for _ in range(_NBUF)],
            [pltpu.SemaphoreType.DMA for _ in range(_NBUF)],
            [pltpu.SemaphoreType.DMA for _ in range(2)],
        ],
    )


def _spmm_call(x, edge2, zero_rows):
    return _get_spmm_call()(x, edge2, zero_rows)


def _mlp_body(x_ref, a0_ref, a1_ref, w1_ref, b1_ref, w2_ref, b2_ref, o_ref):
    h = (x_ref[...] + a0_ref[...].reshape(_BR, _D)
         + a1_ref[...].reshape(_BR, _D))
    h = jnp.maximum(
        jnp.dot(h, w1_ref[...], preferred_element_type=jnp.float32)
        + b1_ref[...], 0.0)
    h = jnp.maximum(
        jnp.dot(h, w2_ref[...], preferred_element_type=jnp.float32)
        + b2_ref[...], 0.0)
    o_ref[...] = h


_BR = 2000
_NB = _N // _BR


def _mlp_tc(x, agg, w1, b1, w2, b2):
    row_spec = pl.BlockSpec((_BR, _D), lambda i: (i, 0))
    a0_spec = pl.BlockSpec((1, _BR, _D), lambda i: (0, i, 0))
    a1_spec = pl.BlockSpec((1, _BR, _D), lambda i: (1, i, 0))
    w_spec = pl.BlockSpec((_D, _D), lambda i: (0, 0))
    b_spec = pl.BlockSpec((1, _D), lambda i: (0, 0))
    return pl.pallas_call(
        _mlp_body,
        grid=(_NB,),
        in_specs=[row_spec, a0_spec, a1_spec, w_spec, b_spec, w_spec, b_spec],
        out_specs=row_spec,
        out_shape=jax.ShapeDtypeStruct((_N, _D), jnp.float32),
    )(x, agg, agg, w1, b1, w2, b2)


def _final_body(x_ref, a0_ref, a1_ref, w1_ref, b1_ref, w2_ref, b2_ref,
                bt_ref, wfc_ref, bfc_ref, o_ref, p1_ref, p2_ref):
    i = pl.program_id(0)

    @pl.when(i == 0)
    def _():
        p1_ref[...] = jnp.zeros_like(p1_ref)
        p2_ref[...] = jnp.zeros_like(p2_ref)

    x1b = x_ref[...]
    h = (x1b + a0_ref[...].reshape(_BR, _D)
         + a1_ref[...].reshape(_BR, _D))
    h = jnp.maximum(
        jnp.dot(h, w1_ref[...], preferred_element_type=jnp.float32)
        + b1_ref[...], 0.0)
    x2b = jnp.maximum(
        jnp.dot(h, w2_ref[...], preferred_element_type=jnp.float32)
        + b2_ref[...], 0.0)

    b2d = bt_ref[...].reshape(1, _BR)
    gids = lax.broadcasted_iota(jnp.int32, (_G, _BR), 0)
    pt = (gids == b2d).astype(jnp.float32)
    p1_ref[...] += jnp.dot(pt, x1b, preferred_element_type=jnp.float32)
    p2_ref[...] += jnp.dot(pt, x2b, preferred_element_type=jnp.float32)

    @pl.when(i == _NB - 1)
    def _():
        pooled = (
            jnp.dot(p1_ref[...], wfc_ref[0:_D, :],
                    preferred_element_type=jnp.float32)
            + jnp.dot(p2_ref[...], wfc_ref[_D:2 * _D, :],
                      preferred_element_type=jnp.float32)
            + bfc_ref[...])
        m = jnp.max(pooled, axis=-1, keepdims=True)
        lse = jnp.log(jnp.sum(jnp.exp(pooled - m), axis=-1, keepdims=True)) + m
        o_ref[...] = pooled - lse


def _final_tc(x1, agg, w1, b1, w2, b2, batch3, wfc, bfc):
    row_spec = pl.BlockSpec((_BR, _D), lambda i: (i, 0))
    a0_spec = pl.BlockSpec((1, _BR, _D), lambda i: (0, i, 0))
    a1_spec = pl.BlockSpec((1, _BR, _D), lambda i: (1, i, 0))
    w_spec = pl.BlockSpec((_D, _D), lambda i: (0, 0))
    b_spec = pl.BlockSpec((1, _D), lambda i: (0, 0))
    return pl.pallas_call(
        _final_body,
        grid=(_NB,),
        in_specs=[
            row_spec, a0_spec, a1_spec, w_spec, b_spec, w_spec, b_spec,
            pl.BlockSpec((1, 1, _BR), lambda i: (i, 0, 0)),
            pl.BlockSpec((2 * _D, _D), lambda i: (0, 0)),
            b_spec,
        ],
        out_specs=pl.BlockSpec((_G, _D), lambda i: (0, 0)),
        out_shape=jax.ShapeDtypeStruct((_G, _D), jnp.float32),
        scratch_shapes=[
            pltpu.VMEM((_G, _D), jnp.float32),
            pltpu.VMEM((_G, _D), jnp.float32),
        ],
    )(x1, agg, agg, w1, b1, w2, b2, batch3, wfc, bfc)


def kernel(x, edge_index, batch, W1_0, b1_0, W2_0, b2_0,
           W1_1, b1_1, W2_1, b2_1, Wfc, bfc):
    zero_rows = jnp.zeros((_N + _PADROWS, _D), jnp.float32)

    b1_0r = b1_0.reshape(1, _D)
    b2_0r = b2_0.reshape(1, _D)
    b1_1r = b1_1.reshape(1, _D)
    b2_1r = b2_1.reshape(1, _D)
    bfcr = bfc.reshape(1, _D)
    batch3 = batch.reshape(_NB, 1, _BR)

    agg1 = _spmm_call(x, edge_index, zero_rows)
    x1 = _mlp_tc(x, agg1, W1_0, b1_0r, W2_0, b2_0r)
    agg2 = _spmm_call(x1, edge_index, zero_rows)
    out = _final_tc(x1, agg2, W1_1, b1_1r, W2_1, b2_1r, batch3, Wfc, bfcr)
    return out


# speedup vs baseline: 1.1655x; 1.0240x over previous
import functools

import jax
import jax.numpy as jnp
from jax import lax
from jax.experimental import pallas as pl
from jax.experimental.pallas import tpu as pltpu
from jax.experimental.pallas import tpu_sc as plsc

_N = 10000
_E = 320000
_D = 128
_G = 64
_NC = 2
_NS = 16
_NW = _NC * _NS
_CH = 40
_EPT = 10000
_RPT = 624
_LAST = _N - _RPT * (_NS - 1)
_PADROWS = 16


_NCHUNK = _EPT // _CH
_NBUF = 5
_SEG = 50
_NSEG = _NCHUNK // _SEG


def _spmm_body(x_hbm, edge_hbm, out_hbm,
               sidx, didx, rows, agg, semg, sems, semi):
    cid = lax.axis_index("c")
    sid = lax.axis_index("s")
    wid = cid * _NS + sid
    base = sid * _RPT
    src_hbm = edge_hbm.at[0]
    dst_hbm = edge_hbm.at[1]

    def zstep(k, carry):
        rows[0][k // 8, pl.ds((k % 8) * 16, 16)] = jnp.zeros((16,), jnp.float32)
        return carry

    lax.fori_loop(0, _CH * 8, zstep, 0)

    def zcopy(off, n):
        pltpu.async_copy(rows[0].at[pl.ds(0, n)],
                         agg.at[pl.ds(base + off, n)], semg[0])

    def zcopy_wait(n):
        pltpu.make_async_copy(rows[0].at[pl.ds(0, n)],
                              agg.at[pl.ds(base, n)], semg[0]).wait()

    _zs = []
    for k in range(_RPT // _CH):
        _zs.append((k * _CH, _CH))
    _zs.append((_RPT - _RPT % _CH if _RPT % _CH else _RPT, _RPT % _CH))
    _zs = [(o, n) for o, n in _zs if n > 0]

    for o, n in _zs:
        zcopy(o, n)

    @pl.when(sid == _NS - 1)
    def _():
        extra = _LAST + _PADROWS - _RPT
        pltpu.async_copy(rows[0].at[pl.ds(0, extra)],
                         agg.at[pl.ds(base + _RPT, extra)], semg[0])

    for _, n in _zs:
        zcopy_wait(n)

    @pl.when(sid == _NS - 1)
    def _():
        zcopy_wait(_LAST + _PADROWS - _RPT)

    plsc.subcore_barrier()

    def gather(k, b, s):
        pltpu.async_copy(x_hbm.at[sidx[s].at[pl.ds(k * _CH, _CH)]],
                         rows[b], semg[b])

    def gather_wait(b):
        pltpu.make_async_copy(x_hbm.at[sidx[0].at[pl.ds(0, _CH)]],
                              rows[b], semg[b]).wait()

    def scatter(k, b, s):
        pltpu.async_copy(rows[b], agg.at[didx[s].at[pl.ds(k * _CH, _CH)]],
                         sems[b], add=True)

    def scatter_wait(b):
        pltpu.make_async_copy(rows[b], agg.at[didx[0].at[pl.ds(0, _CH)]],
                              sems[b]).wait()

    def load_idx(seg, s):
        segbase = wid * _EPT + seg * _SEG * _CH
        pltpu.async_copy(src_hbm.at[pl.ds(segbase, _SEG * _CH)],
                         sidx[s], semi[s])
        pltpu.async_copy(dst_hbm.at[pl.ds(segbase, _SEG * _CH)],
                         didx[s], semi[s])

    def wait_idx(s):
        pltpu.make_async_copy(src_hbm.at[pl.ds(0, _SEG * _CH)],
                              sidx[s], semi[s]).wait()
        pltpu.make_async_copy(dst_hbm.at[pl.ds(0, _SEG * _CH)],
                              didx[s], semi[s]).wait()

    _NITER = _SEG // _NBUF

    load_idx(0, 0)
    wait_idx(0)
    for b in range(_NBUF):
        gather(b, b, 0)

    for seg in range(_NSEG):
        s = seg % 2
        ns = (seg + 1) % 2
        if seg + 1 < _NSEG:
            load_idx(seg + 1, ns)

        def step(i, carry):
            k0 = i * _NBUF
            for b in range(_NBUF):
                gather_wait(b)
                scatter(k0 + b, b, s)
            for b in range(_NBUF):
                @pl.when(i < _NITER - 1)
                def _(b=b):
                    scatter_wait(b)
                    gather(k0 + _NBUF + b, b, s)
            return carry

        lax.fori_loop(0, _NITER, step, 0)

        if seg + 1 < _NSEG:
            wait_idx(ns)
            for b in range(_NBUF):
                scatter_wait(b)
                gather(b, b, ns)
        else:
            for b in range(_NBUF):
                scatter_wait(b)

    plsc.subcore_barrier()

    @pl.when(jnp.logical_and(cid == 0, sid < _NS - 1))
    def _():
        pltpu.sync_copy(agg.at[pl.ds(base, _RPT)],
                        out_hbm.at[0].at[pl.ds(base, _RPT)])

    @pl.when(jnp.logical_and(cid == 0, sid == _NS - 1))
    def _():
        pltpu.sync_copy(agg.at[pl.ds(_RPT * (_NS - 1), _LAST)],
                        out_hbm.at[0].at[pl.ds(_RPT * (_NS - 1), _LAST)])

    @pl.when(jnp.logical_and(cid == 1, sid < _NS - 1))
    def _():
        pltpu.sync_copy(agg.at[pl.ds(base, _RPT)],
                        out_hbm.at[1].at[pl.ds(base, _RPT)])

    @pl.when(jnp.logical_and(cid == 1, sid == _NS - 1))
    def _():
        pltpu.sync_copy(agg.at[pl.ds(_RPT * (_NS - 1), _LAST)],
                        out_hbm.at[1].at[pl.ds(_RPT * (_NS - 1), _LAST)])


@functools.cache
def _get_spmm_call():
    return pl.kernel(
        _spmm_body,
        out_type=jax.ShapeDtypeStruct((_NC, _N, _D), jnp.float32),
        mesh=plsc.VectorSubcoreMesh(core_axis_name="c", subcore_axis_name="s",
                                    num_cores=_NC, num_subcores=_NS),
        compiler_params=pltpu.CompilerParams(use_tc_tiling_on_sc=False),
        scratch_types=[
            [pltpu.VMEM((_SEG * _CH,), jnp.int32) for _ in range(2)],
            [pltpu.VMEM((_SEG * _CH,), jnp.int32) for _ in range(2)],
            [pltpu.VMEM((_CH, _D), jnp.float32) for _ in range(_NBUF)],
            pltpu.VMEM_SHARED((_N + _PADROWS, _D), jnp.float32),
            [pltpu.SemaphoreType.DMA for _ in range(_NBUF)],
            [pltpu.SemaphoreType.DMA for _ in range(_NBUF)],
            [pltpu.SemaphoreType.DMA for _ in range(2)],
        ],
    )


def _spmm_call(x, edge2):
    return _get_spmm_call()(x, edge2)


def _mlp_body(x_ref, a0_ref, a1_ref, w1_ref, b1_ref, w2_ref, b2_ref, o_ref):
    h = (x_ref[...] + a0_ref[...].reshape(_BR, _D)
         + a1_ref[...].reshape(_BR, _D))
    h = jnp.maximum(
        jnp.dot(h, w1_ref[...], preferred_element_type=jnp.float32)
        + b1_ref[...], 0.0)
    h = jnp.maximum(
        jnp.dot(h, w2_ref[...], preferred_element_type=jnp.float32)
        + b2_ref[...], 0.0)
    o_ref[...] = h


_BR = 2000
_NB = _N // _BR


def _mlp_tc(x, agg, w1, b1, w2, b2):
    row_spec = pl.BlockSpec((_BR, _D), lambda i: (i, 0))
    a0_spec = pl.BlockSpec((1, _BR, _D), lambda i: (0, i, 0))
    a1_spec = pl.BlockSpec((1, _BR, _D), lambda i: (1, i, 0))
    w_spec = pl.BlockSpec((_D, _D), lambda i: (0, 0))
    b_spec = pl.BlockSpec((1, _D), lambda i: (0, 0))
    return pl.pallas_call(
        _mlp_body,
        grid=(_NB,),
        in_specs=[row_spec, a0_spec, a1_spec, w_spec, b_spec, w_spec, b_spec],
        out_specs=row_spec,
        out_shape=jax.ShapeDtypeStruct((_N, _D), jnp.float32),
    )(x, agg, agg, w1, b1, w2, b2)


def _final_body(x_ref, a0_ref, a1_ref, w1_ref, b1_ref, w2_ref, b2_ref,
                bt_ref, wfc_ref, bfc_ref, o_ref, p1_ref, p2_ref):
    i = pl.program_id(0)

    @pl.when(i == 0)
    def _():
        p1_ref[...] = jnp.zeros_like(p1_ref)
        p2_ref[...] = jnp.zeros_like(p2_ref)

    x1b = x_ref[...]
    h = (x1b + a0_ref[...].reshape(_BR, _D)
         + a1_ref[...].reshape(_BR, _D))
    h = jnp.maximum(
        jnp.dot(h, w1_ref[...], preferred_element_type=jnp.float32)
        + b1_ref[...], 0.0)
    x2b = jnp.maximum(
        jnp.dot(h, w2_ref[...], preferred_element_type=jnp.float32)
        + b2_ref[...], 0.0)

    b2d = bt_ref[...].reshape(1, _BR)
    gids = lax.broadcasted_iota(jnp.int32, (_G, _BR), 0)
    pt = (gids == b2d).astype(jnp.float32)
    p1_ref[...] += jnp.dot(pt, x1b, preferred_element_type=jnp.float32)
    p2_ref[...] += jnp.dot(pt, x2b, preferred_element_type=jnp.float32)

    @pl.when(i == _NB - 1)
    def _():
        pooled = (
            jnp.dot(p1_ref[...], wfc_ref[0:_D, :],
                    preferred_element_type=jnp.float32)
            + jnp.dot(p2_ref[...], wfc_ref[_D:2 * _D, :],
                      preferred_element_type=jnp.float32)
            + bfc_ref[...])
        m = jnp.max(pooled, axis=-1, keepdims=True)
        lse = jnp.log(jnp.sum(jnp.exp(pooled - m), axis=-1, keepdims=True)) + m
        o_ref[...] = pooled - lse


def _final_tc(x1, agg, w1, b1, w2, b2, batch3, wfc, bfc):
    row_spec = pl.BlockSpec((_BR, _D), lambda i: (i, 0))
    a0_spec = pl.BlockSpec((1, _BR, _D), lambda i: (0, i, 0))
    a1_spec = pl.BlockSpec((1, _BR, _D), lambda i: (1, i, 0))
    w_spec = pl.BlockSpec((_D, _D), lambda i: (0, 0))
    b_spec = pl.BlockSpec((1, _D), lambda i: (0, 0))
    return pl.pallas_call(
        _final_body,
        grid=(_NB,),
        in_specs=[
            row_spec, a0_spec, a1_spec, w_spec, b_spec, w_spec, b_spec,
            pl.BlockSpec((1, 1, _BR), lambda i: (i, 0, 0)),
            pl.BlockSpec((2 * _D, _D), lambda i: (0, 0)),
            b_spec,
        ],
        out_specs=pl.BlockSpec((_G, _D), lambda i: (0, 0)),
        out_shape=jax.ShapeDtypeStruct((_G, _D), jnp.float32),
        scratch_shapes=[
            pltpu.VMEM((_G, _D), jnp.float32),
            pltpu.VMEM((_G, _D), jnp.float32),
        ],
    )(x1, agg, agg, w1, b1, w2, b2, batch3, wfc, bfc)


def kernel(x, edge_index, batch, W1_0, b1_0, W2_0, b2_0,
           W1_1, b1_1, W2_1, b2_1, Wfc, bfc):

    b1_0r = b1_0.reshape(1, _D)
    b2_0r = b2_0.reshape(1, _D)
    b1_1r = b1_1.reshape(1, _D)
    b2_1r = b2_1.reshape(1, _D)
    bfcr = bfc.reshape(1, _D)
    batch3 = batch.reshape(_NB, 1, _BR)

    agg1 = _spmm_call(x, edge_index)
    x1 = _mlp_tc(x, agg1, W1_0, b1_0r, W2_0, b2_0r)
    agg2 = _spmm_call(x1, edge_index)
    out = _final_tc(x1, agg2, W1_1, b1_1r, W2_1, b2_1r, batch3, Wfc, bfcr)
    return out


# speedup vs baseline: 1.1718x; 1.0054x over previous
import functools

import jax
import jax.numpy as jnp
from jax import lax
from jax.experimental import pallas as pl
from jax.experimental.pallas import tpu as pltpu
from jax.experimental.pallas import tpu_sc as plsc

_N = 10000
_E = 320000
_D = 128
_G = 64
_NC = 2
_NS = 16
_NW = _NC * _NS
_CH = 40
_EPT = 10000
_RPT = 624
_LAST = _N - _RPT * (_NS - 1)
_PADROWS = 16


_NCHUNK = _EPT // _CH
_NBUF = 5
_SEG = 125
_NSEG = _NCHUNK // _SEG


def _spmm_body(x_hbm, edge_hbm, out_hbm,
               sidx, didx, rows, agg, semg, sems, semi):
    cid = lax.axis_index("c")
    sid = lax.axis_index("s")
    wid = cid * _NS + sid
    base = sid * _RPT
    src_hbm = edge_hbm.at[0]
    dst_hbm = edge_hbm.at[1]

    def zstep(k, carry):
        rows[0][k // 8, pl.ds((k % 8) * 16, 16)] = jnp.zeros((16,), jnp.float32)
        return carry

    lax.fori_loop(0, _CH * 8, zstep, 0)

    def zcopy(off, n):
        pltpu.async_copy(rows[0].at[pl.ds(0, n)],
                         agg.at[pl.ds(base + off, n)], semg[0])

    def zcopy_wait(n):
        pltpu.make_async_copy(rows[0].at[pl.ds(0, n)],
                              agg.at[pl.ds(base, n)], semg[0]).wait()

    _zs = []
    for k in range(_RPT // _CH):
        _zs.append((k * _CH, _CH))
    _zs.append((_RPT - _RPT % _CH if _RPT % _CH else _RPT, _RPT % _CH))
    _zs = [(o, n) for o, n in _zs if n > 0]

    for o, n in _zs:
        zcopy(o, n)

    @pl.when(sid == _NS - 1)
    def _():
        extra = _LAST + _PADROWS - _RPT
        pltpu.async_copy(rows[0].at[pl.ds(0, extra)],
                         agg.at[pl.ds(base + _RPT, extra)], semg[0])

    for _, n in _zs:
        zcopy_wait(n)

    @pl.when(sid == _NS - 1)
    def _():
        zcopy_wait(_LAST + _PADROWS - _RPT)

    plsc.subcore_barrier()

    def gather(k, b, s):
        pltpu.async_copy(x_hbm.at[sidx[s].at[pl.ds(k * _CH, _CH)]],
                         rows[b], semg[b])

    def gather_wait(b):
        pltpu.make_async_copy(x_hbm.at[sidx[0].at[pl.ds(0, _CH)]],
                              rows[b], semg[b]).wait()

    def scatter(k, b, s):
        pltpu.async_copy(rows[b], agg.at[didx[s].at[pl.ds(k * _CH, _CH)]],
                         sems[b], add=True)

    def scatter_wait(b):
        pltpu.make_async_copy(rows[b], agg.at[didx[0].at[pl.ds(0, _CH)]],
                              sems[b]).wait()

    def load_idx(seg, s):
        segbase = wid * _EPT + seg * _SEG * _CH
        pltpu.async_copy(src_hbm.at[pl.ds(segbase, _SEG * _CH)],
                         sidx[s], semi[s])
        pltpu.async_copy(dst_hbm.at[pl.ds(segbase, _SEG * _CH)],
                         didx[s], semi[s])

    def wait_idx(s):
        pltpu.make_async_copy(src_hbm.at[pl.ds(0, _SEG * _CH)],
                              sidx[s], semi[s]).wait()
        pltpu.make_async_copy(dst_hbm.at[pl.ds(0, _SEG * _CH)],
                              didx[s], semi[s]).wait()

    _NITER = _SEG // _NBUF

    load_idx(0, 0)
    wait_idx(0)
    for b in range(_NBUF):
        gather(b, b, 0)

    for seg in range(_NSEG):
        s = seg % 2
        ns = (seg + 1) % 2
        if seg + 1 < _NSEG:
            load_idx(seg + 1, ns)

        def step(i, carry):
            k0 = i * _NBUF
            for b in range(_NBUF):
                gather_wait(b)
                scatter(k0 + b, b, s)
            for b in range(_NBUF):
                @pl.when(i < _NITER - 1)
                def _(b=b):
                    scatter_wait(b)
                    gather(k0 + _NBUF + b, b, s)
            return carry

        lax.fori_loop(0, _NITER, step, 0)

        if seg + 1 < _NSEG:
            wait_idx(ns)
            for b in range(_NBUF):
                scatter_wait(b)
                gather(b, b, ns)
        else:
            for b in range(_NBUF):
                scatter_wait(b)

    plsc.subcore_barrier()

    @pl.when(jnp.logical_and(cid == 0, sid < _NS - 1))
    def _():
        pltpu.sync_copy(agg.at[pl.ds(base, _RPT)],
                        out_hbm.at[0].at[pl.ds(base, _RPT)])

    @pl.when(jnp.logical_and(cid == 0, sid == _NS - 1))
    def _():
        pltpu.sync_copy(agg.at[pl.ds(_RPT * (_NS - 1), _LAST)],
                        out_hbm.at[0].at[pl.ds(_RPT * (_NS - 1), _LAST)])

    @pl.when(jnp.logical_and(cid == 1, sid < _NS - 1))
    def _():
        pltpu.sync_copy(agg.at[pl.ds(base, _RPT)],
                        out_hbm.at[1].at[pl.ds(base, _RPT)])

    @pl.when(jnp.logical_and(cid == 1, sid == _NS - 1))
    def _():
        pltpu.sync_copy(agg.at[pl.ds(_RPT * (_NS - 1), _LAST)],
                        out_hbm.at[1].at[pl.ds(_RPT * (_NS - 1), _LAST)])


@functools.cache
def _get_spmm_call():
    return pl.kernel(
        _spmm_body,
        out_type=jax.ShapeDtypeStruct((_NC, _N, _D), jnp.float32),
        mesh=plsc.VectorSubcoreMesh(core_axis_name="c", subcore_axis_name="s",
                                    num_cores=_NC, num_subcores=_NS),
        compiler_params=pltpu.CompilerParams(use_tc_tiling_on_sc=False),
        scratch_types=[
            [pltpu.VMEM((_SEG * _CH,), jnp.int32) for _ in range(2)],
            [pltpu.VMEM((_SEG * _CH,), jnp.int32) for _ in range(2)],
            [pltpu.VMEM((_CH, _D), jnp.float32) for _ in range(_NBUF)],
            pltpu.VMEM_SHARED((_N + _PADROWS, _D), jnp.float32),
            [pltpu.SemaphoreType.DMA for _ in range(_NBUF)],
            [pltpu.SemaphoreType.DMA for _ in range(_NBUF)],
            [pltpu.SemaphoreType.DMA for _ in range(2)],
        ],
    )


def _spmm_call(x, edge2):
    return _get_spmm_call()(x, edge2)


def _mlp_body(x_ref, a0_ref, a1_ref, w1_ref, b1_ref, w2_ref, b2_ref, o_ref):
    h = (x_ref[...] + a0_ref[...].reshape(_BR, _D)
         + a1_ref[...].reshape(_BR, _D))
    h = jnp.maximum(
        jnp.dot(h, w1_ref[...], preferred_element_type=jnp.float32)
        + b1_ref[...], 0.0)
    h = jnp.maximum(
        jnp.dot(h, w2_ref[...], preferred_element_type=jnp.float32)
        + b2_ref[...], 0.0)
    o_ref[...] = h


_BR = 2000
_NB = _N // _BR


def _mlp_tc(x, agg, w1, b1, w2, b2):
    row_spec = pl.BlockSpec((_BR, _D), lambda i: (i, 0))
    a0_spec = pl.BlockSpec((1, _BR, _D), lambda i: (0, i, 0))
    a1_spec = pl.BlockSpec((1, _BR, _D), lambda i: (1, i, 0))
    w_spec = pl.BlockSpec((_D, _D), lambda i: (0, 0))
    b_spec = pl.BlockSpec((1, _D), lambda i: (0, 0))
    return pl.pallas_call(
        _mlp_body,
        grid=(_NB,),
        in_specs=[row_spec, a0_spec, a1_spec, w_spec, b_spec, w_spec, b_spec],
        out_specs=row_spec,
        out_shape=jax.ShapeDtypeStruct((_N, _D), jnp.float32),
    )(x, agg, agg, w1, b1, w2, b2)


def _final_body(x_ref, a0_ref, a1_ref, w1_ref, b1_ref, w2_ref, b2_ref,
                bt_ref, wfc_ref, bfc_ref, o_ref, p1_ref, p2_ref):
    i = pl.program_id(0)

    @pl.when(i == 0)
    def _():
        p1_ref[...] = jnp.zeros_like(p1_ref)
        p2_ref[...] = jnp.zeros_like(p2_ref)

    x1b = x_ref[...]
    h = (x1b + a0_ref[...].reshape(_BR, _D)
         + a1_ref[...].reshape(_BR, _D))
    h = jnp.maximum(
        jnp.dot(h, w1_ref[...], preferred_element_type=jnp.float32)
        + b1_ref[...], 0.0)
    x2b = jnp.maximum(
        jnp.dot(h, w2_ref[...], preferred_element_type=jnp.float32)
        + b2_ref[...], 0.0)

    b2d = bt_ref[...].reshape(1, _BR)
    gids = lax.broadcasted_iota(jnp.int32, (_G, _BR), 0)
    pt = (gids == b2d).astype(jnp.float32)
    p1_ref[...] += jnp.dot(pt, x1b, preferred_element_type=jnp.float32)
    p2_ref[...] += jnp.dot(pt, x2b, preferred_element_type=jnp.float32)

    @pl.when(i == _NB - 1)
    def _():
        pooled = (
            jnp.dot(p1_ref[...], wfc_ref[0:_D, :],
                    preferred_element_type=jnp.float32)
            + jnp.dot(p2_ref[...], wfc_ref[_D:2 * _D, :],
                      preferred_element_type=jnp.float32)
            + bfc_ref[...])
        m = jnp.max(pooled, axis=-1, keepdims=True)
        lse = jnp.log(jnp.sum(jnp.exp(pooled - m), axis=-1, keepdims=True)) + m
        o_ref[...] = pooled - lse


def _final_tc(x1, agg, w1, b1, w2, b2, batch3, wfc, bfc):
    row_spec = pl.BlockSpec((_BR, _D), lambda i: (i, 0))
    a0_spec = pl.BlockSpec((1, _BR, _D), lambda i: (0, i, 0))
    a1_spec = pl.BlockSpec((1, _BR, _D), lambda i: (1, i, 0))
    w_spec = pl.BlockSpec((_D, _D), lambda i: (0, 0))
    b_spec = pl.BlockSpec((1, _D), lambda i: (0, 0))
    return pl.pallas_call(
        _final_body,
        grid=(_NB,),
        in_specs=[
            row_spec, a0_spec, a1_spec, w_spec, b_spec, w_spec, b_spec,
            pl.BlockSpec((1, 1, _BR), lambda i: (i, 0, 0)),
            pl.BlockSpec((2 * _D, _D), lambda i: (0, 0)),
            b_spec,
        ],
        out_specs=pl.BlockSpec((_G, _D), lambda i: (0, 0)),
        out_shape=jax.ShapeDtypeStruct((_G, _D), jnp.float32),
        scratch_shapes=[
            pltpu.VMEM((_G, _D), jnp.float32),
            pltpu.VMEM((_G, _D), jnp.float32),
        ],
    )(x1, agg, agg, w1, b1, w2, b2, batch3, wfc, bfc)


def kernel(x, edge_index, batch, W1_0, b1_0, W2_0, b2_0,
           W1_1, b1_1, W2_1, b2_1, Wfc, bfc):

    b1_0r = b1_0.reshape(1, _D)
    b2_0r = b2_0.reshape(1, _D)
    b1_1r = b1_1.reshape(1, _D)
    b2_1r = b2_1.reshape(1, _D)
    bfcr = bfc.reshape(1, _D)
    batch3 = batch.reshape(_NB, 1, _BR)

    agg1 = _spmm_call(x, edge_index)
    x1 = _mlp_tc(x, agg1, W1_0, b1_0r, W2_0, b2_0r)
    agg2 = _spmm_call(x1, edge_index)
    out = _final_tc(x1, agg2, W1_1, b1_1r, W2_1, b2_1r, batch3, Wfc, bfcr)
    return out
